# baseline (speedup 1.0000x reference)
import functools

import jax
import jax.numpy as jnp
from jax import lax
from jax.experimental import pallas as pl
from jax.experimental.pallas import tpu as pltpu
from jax.experimental.pallas import tpu_sc as plsc

D = 128
N_NODES = 10000
N_EDGES = 320000

NC = 2
NS = 16
NW = NC * NS
LANES = 16

N_CHUNKS = N_EDGES // 128
CPW = -(-N_CHUNKS // NW)
EPW = CPW * 128


N_PAD = 10240
ROW_BLK = 2048


def _pq_tc_kernel(x_ref, w_ref, b_ref, p_ref, q_ref):
    pq = jax.lax.dot_general(
        w_ref[...], x_ref[...], (((1,), (1,)), ((), ())),
        preferred_element_type=jnp.float32,
    )
    p_ref[...] = pq[0, :] + b_ref[0]
    q_ref[...] = pq[1, :]


def _compute_pq(x, w2, b):
    return pl.pallas_call(
        _pq_tc_kernel,
        grid=(N_PAD // ROW_BLK,),
        compiler_params=pltpu.CompilerParams(
            vmem_limit_bytes=4 * 1024 * 1024,
        ),
        in_specs=[
            pl.BlockSpec((ROW_BLK, D), lambda i: (i, 0)),
            pl.BlockSpec((2, D), lambda i: (0, 0)),
            pl.BlockSpec(memory_space=pltpu.SMEM),
        ],
        out_specs=[
            pl.BlockSpec((ROW_BLK,), lambda i: (i,)),
            pl.BlockSpec((ROW_BLK,), lambda i: (i,)),
        ],
        out_shape=[
            jax.ShapeDtypeStruct((N_PAD,), jnp.float32),
            jax.ShapeDtypeStruct((N_PAD,), jnp.float32),
        ],
    )(x, w2, b)


@functools.partial(
    pl.kernel,
    out_type=jax.ShapeDtypeStruct((N_EDGES,), jnp.float32),
    mesh=plsc.VectorSubcoreMesh(core_axis_name="c", subcore_axis_name="s"),
    compiler_params=pltpu.CompilerParams(
        needs_layout_passes=False,
        disable_bounds_checks=True,
        disable_semaphore_checks=True,
    ),
    scratch_types=[
        pltpu.VMEM((N_PAD,), jnp.float32),
        pltpu.VMEM((N_PAD,), jnp.float32),
        pltpu.VMEM((CPW, 2, 128), jnp.int32),
        pltpu.VMEM((EPW,), jnp.float32),
        pltpu.SemaphoreType.DMA,
        pltpu.SemaphoreType.DMA,
        pltpu.SemaphoreType.DMA,
    ],
)
def _sc_edge_kernel(p_hbm, q_hbm, idx_hbm, out_hbm,
                    p_v, q_v, idx_v, out_v, sem0, sem1, sem2):
    wid = lax.axis_index("s") * NC + lax.axis_index("c")
    base_c = jnp.minimum(wid * CPW, N_CHUNKS - CPW)
    c0 = pltpu.async_copy(idx_hbm.at[pl.ds(base_c, CPW)], idx_v, sem0)
    c1 = pltpu.async_copy(p_hbm, p_v, sem1)
    c2 = pltpu.async_copy(q_hbm, q_v, sem2)
    c1.wait()
    c2.wait()
    c0.wait()

    @plsc.parallel_loop(0, CPW * 8, 1, unroll=8)
    def body(k):
        c = k >> 3
        off = (k & 7) * LANES
        si = idx_v[c, 0, pl.ds(off, LANES)]
        di = idx_v[c, 1, pl.ds(off, LANES)]
        vp = plsc.load_gather(p_v, [si])
        vq = plsc.load_gather(q_v, [di])
        out_v[pl.ds(k * LANES, LANES)] = vp + vq

    pltpu.sync_copy(out_v, out_hbm.at[pl.ds(base_c * 128, EPW)])


def kernel(x, edge_index, W, b):
    w2 = W[:, 0].reshape(2, D)
    p, q = _compute_pq(x, w2, b)
    ei = edge_index.astype(jnp.int32)
    idx3 = ei.reshape(2, N_CHUNKS, 128).transpose(1, 0, 2)
    out = _sc_edge_kernel(p, q, idx3)
    return out.reshape(N_EDGES, 1)


# speedup vs baseline: 1.0465x; 1.0465x over previous
import functools

import jax
import jax.numpy as jnp
from jax import lax
from jax.experimental import pallas as pl
from jax.experimental.pallas import tpu as pltpu
from jax.experimental.pallas import tpu_sc as plsc

D = 128
N_NODES = 10000
N_EDGES = 320000

NC = 2
NS = 16
NW = NC * NS
LANES = 16

N_CHUNKS = N_EDGES // 128
CPW = -(-N_CHUNKS // NW)
EPW = CPW * 128


def _pq_tc_kernel(x_ref, w_ref, b_ref, p_ref, q_ref):
    pq = jax.lax.dot_general(
        w_ref[...], x_ref[...], (((1,), (1,)), ((), ())),
        preferred_element_type=jnp.float32,
    )
    p_ref[...] = pq[0, :] + b_ref[0]
    q_ref[...] = pq[1, :]


def _compute_pq(x, w2, b):
    n = x.shape[0]
    return pl.pallas_call(
        _pq_tc_kernel,
        in_specs=[
            pl.BlockSpec((n, D), lambda: (0, 0)),
            pl.BlockSpec((2, D), lambda: (0, 0)),
            pl.BlockSpec(memory_space=pltpu.SMEM),
        ],
        out_specs=[
            pl.BlockSpec((n,), lambda: (0,)),
            pl.BlockSpec((n,), lambda: (0,)),
        ],
        out_shape=[
            jax.ShapeDtypeStruct((n,), jnp.float32),
            jax.ShapeDtypeStruct((n,), jnp.float32),
        ],
    )(x, w2, b)


@functools.partial(
    pl.kernel,
    out_type=jax.ShapeDtypeStruct((N_EDGES,), jnp.float32),
    mesh=plsc.VectorSubcoreMesh(core_axis_name="c", subcore_axis_name="s"),
    compiler_params=pltpu.CompilerParams(
        needs_layout_passes=False,
        disable_bounds_checks=True,
        disable_semaphore_checks=True,
    ),
    scratch_types=[
        pltpu.VMEM((N_NODES,), jnp.float32),
        pltpu.VMEM((N_NODES,), jnp.float32),
        pltpu.VMEM((CPW, 2, 128), jnp.int32),
        pltpu.VMEM((EPW,), jnp.float32),
        pltpu.SemaphoreType.DMA,
        pltpu.SemaphoreType.DMA,
        pltpu.SemaphoreType.DMA,
    ],
)
def _sc_edge_kernel(p_hbm, q_hbm, idx_hbm, out_hbm,
                    p_v, q_v, idx_v, out_v, sem0, sem1, sem2):
    wid = lax.axis_index("s") * NC + lax.axis_index("c")
    base_c = jnp.minimum(wid * CPW, N_CHUNKS - CPW)
    c0 = pltpu.async_copy(idx_hbm.at[pl.ds(base_c, CPW)], idx_v, sem0)
    c1 = pltpu.async_copy(p_hbm, p_v, sem1)
    c2 = pltpu.async_copy(q_hbm, q_v, sem2)
    c1.wait()
    c2.wait()
    c0.wait()

    @plsc.parallel_loop(0, CPW * 8, 1, unroll=8)
    def body(k):
        c = k >> 3
        off = (k & 7) * LANES
        si = idx_v[c, 0, pl.ds(off, LANES)]
        di = idx_v[c, 1, pl.ds(off, LANES)]
        vp = plsc.load_gather(p_v, [si])
        vq = plsc.load_gather(q_v, [di])
        out_v[pl.ds(k * LANES, LANES)] = vp + vq

    pltpu.sync_copy(out_v, out_hbm.at[pl.ds(base_c * 128, EPW)])


def kernel(x, edge_index, W, b):
    w2 = W[:, 0].reshape(2, D)
    p, q = _compute_pq(x, w2, b)
    ei = edge_index.astype(jnp.int32)
    idx3 = ei.reshape(2, N_CHUNKS, 128).transpose(1, 0, 2)
    out = _sc_edge_kernel(p, q, idx3)
    return out.reshape(N_EDGES, 1)
